# Initial kernel scaffold; baseline (speedup 1.0000x reference)
import jax
import jax.numpy as jnp
from jax.experimental import pallas as pl


def kernel(x, h0, params):
    raise NotImplementedError("write your pallas kernel here")


# speedup vs baseline: 1.0008x; 1.0008x over previous
import functools

import jax
import jax.numpy as jnp
from jax import lax
from jax.experimental import pallas as pl

_BLOCKS = [2, 2, 6, 2]
_KNN = 9
_MAX_DIL = 49 // _KNN
_EPS = 1e-5


def _layer_specs():
    specs = []
    idx = 0
    hs = [56, 28, 14, 7]
    for i in range(4):
        if i > 0:
            specs.append(("down",))
        for _ in range(_BLOCKS[i]):
            dil = min(idx // 4 + 1, _MAX_DIL)
            specs.append(("grapher", _KNN, dil, [4, 2, 1, 1][i], hs[i]))
            specs.append(("ffn",))
            idx += 1
    return specs


def _topk_body(score_ref, o_ref, *, k, dil):
    score = score_ref[0]
    n, ny = score.shape
    lane = lax.broadcasted_iota(jnp.int32, (n, ny), 1).astype(jnp.float32)
    neg_inf = jnp.float32(-jnp.inf)
    cols = []
    for m in range(k * dil):
        cmax = jnp.max(score, axis=1, keepdims=True)
        hit = score == cmax
        idx = jnp.min(jnp.where(hit, lane, jnp.float32(ny)), axis=1,
                      keepdims=True)
        if m % dil == 0:
            cols.append(idx)
        if m < k * dil - 1:
            score = jnp.where(lane == idx, neg_inf, score)
    o_ref[0] = jnp.concatenate(cols, axis=1).astype(jnp.int32)


def _knn_select(score, k, dil):
    b, n, ny = score.shape
    return pl.pallas_call(
        functools.partial(_topk_body, k=k, dil=dil),
        grid=(b,),
        in_specs=[pl.BlockSpec((1, n, ny), lambda i: (i, 0, 0))],
        out_specs=pl.BlockSpec((1, n, k), lambda i: (i, 0, 0)),
        out_shape=jax.ShapeDtypeStruct((b, n, k), jnp.int32),
    )(score)


def _bn(x, g, b, eps=_EPS):
    mean = x.mean(axis=(0, 2, 3), keepdims=True)
    var = x.var(axis=(0, 2, 3), keepdims=True)
    xn = (x - mean) / jnp.sqrt(var + eps)
    return xn * g[None, :, None, None] + b[None, :, None, None]


def _conv1x1(x, w, b):
    return jnp.einsum("bihw,oi->bohw", x, w) + b[None, :, None, None]


def _down_blk(x, p):
    out = lax.conv_general_dilated(x, p["w"], (2, 2), ((1, 1), (1, 1)),
                                   dimension_numbers=("NCHW", "OIHW", "NCHW"))
    out = out + p["b"][None, :, None, None]
    return _bn(out, p["g"], p["bt"])


def _ffn_blk(x, p):
    s = x
    x = _bn(_conv1x1(x, p["fc1_w"], p["fc1_b"]), p["fc1_g"], p["fc1_bt"])
    x = jax.nn.relu(x)
    x = _bn(_conv1x1(x, p["fc2_w"], p["fc2_b"]), p["fc2_g"], p["fc2_bt"])
    return x + s


def _grapher_blk(x, p, k, dil, r, use_pallas):
    b, c, h, w = x.shape
    shortcut = x
    x = _bn(_conv1x1(x, p["fc1_w"], p["fc1_b"]), p["fc1_g"], p["fc1_bt"])
    n = h * w
    xf = x.reshape(b, c, n)
    if r > 1:
        y = x.reshape(b, c, h // r, r, w // r, r).mean(axis=(3, 5)).reshape(b, c, -1)
    else:
        y = xf
    xn = xf / (jnp.linalg.norm(xf, axis=1, keepdims=True) + 1e-12)
    yn = y / (jnp.linalg.norm(y, axis=1, keepdims=True) + 1e-12)
    dist = (jnp.sum(xn * xn, axis=1)[:, :, None]
            - 2.0 * jnp.einsum("bcn,bcm->bnm", xn, yn)
            + jnp.sum(yn * yn, axis=1)[:, None, :])
    score = -(dist + p["relpos"][None, :, :])
    if use_pallas:
        nn_idx = _knn_select(score, k, dil)
    else:
        _, nn_idx = jax.lax.top_k(score, k * dil)
        nn_idx = nn_idx[:, :, ::dil]
    idx = jnp.broadcast_to(nn_idx.reshape(b, 1, n * k), (b, c, n * k))
    xj = jnp.take_along_axis(y, idx, axis=2).reshape(b, c, n, k)
    xi = xf[:, :, :, None]
    m = jnp.max(xj - xi, axis=3, keepdims=True)
    z = jnp.concatenate([xi[:, :, None, :, :], m[:, :, None, :, :]],
                        axis=2).reshape(b, 2 * c, n, 1)
    z = _conv1x1(z, p["nn_w"], p["nn_b"])
    z = _bn(z, p["nn_g"], p["nn_bt"])
    z = jax.nn.relu(z)
    z = z.reshape(b, 2 * c, h, w)
    z = _bn(_conv1x1(z, p["fc2_w"], p["fc2_b"]), p["fc2_g"], p["fc2_bt"])
    return z + shortcut


def kernel(x, h0, params):
    del h0
    specs = _layer_specs()
    n_b = _BLOCKS
    out_at = [2 * n_b[0] - 1,
              2 * (n_b[0] + n_b[1]),
              2 * (n_b[0] + n_b[1] + n_b[2]) + 1,
              2 * (n_b[0] + n_b[1] + n_b[2] + n_b[3]) + 2]
    outs = []
    gi = 0
    for i, (spec, p) in enumerate(zip(specs, params["layers"])):
        if spec[0] == "down":
            x = _down_blk(x, p)
        elif spec[0] == "grapher":
            x = _grapher_blk(x, p, spec[1], spec[2], spec[3], gi in (0, 1))
            gi += 1
        else:
            x = _ffn_blk(x, p)
        if i in out_at:
            outs.append(x)
    return tuple(outs)


# speedup vs baseline: 1.7058x; 1.7045x over previous
import functools

import jax
import jax.numpy as jnp
from jax import lax
from jax.experimental import pallas as pl

_BLOCKS = [2, 2, 6, 2]
_KNN = 9
_MAX_DIL = 49 // _KNN
_EPS = 1e-5


def _layer_specs():
    specs = []
    idx = 0
    hs = [56, 28, 14, 7]
    for i in range(4):
        if i > 0:
            specs.append(("down",))
        for _ in range(_BLOCKS[i]):
            dil = min(idx // 4 + 1, _MAX_DIL)
            specs.append(("grapher", _KNN, dil, [4, 2, 1, 1][i], hs[i]))
            specs.append(("ffn",))
            idx += 1
    return specs


def _topk_body(score_ref, o_ref, *, k, dil):
    score = score_ref[0]
    n, ny = score.shape
    lane = lax.broadcasted_iota(jnp.int32, (n, ny), 1).astype(jnp.float32)
    neg_inf = jnp.float32(-jnp.inf)
    cols = []
    for m in range(k * dil):
        cmax = jnp.max(score, axis=1, keepdims=True)
        hit = score == cmax
        idx = jnp.min(jnp.where(hit, lane, jnp.float32(ny)), axis=1,
                      keepdims=True)
        if m % dil == 0:
            cols.append(idx)
        if m < k * dil - 1:
            score = jnp.where(lane == idx, neg_inf, score)
    o_ref[0] = jnp.concatenate(cols, axis=1).astype(jnp.int32)


def _knn_select(score, k, dil):
    b, n, ny = score.shape
    return pl.pallas_call(
        functools.partial(_topk_body, k=k, dil=dil),
        grid=(b,),
        in_specs=[pl.BlockSpec((1, n, ny), lambda i: (i, 0, 0))],
        out_specs=pl.BlockSpec((1, n, k), lambda i: (i, 0, 0)),
        out_shape=jax.ShapeDtypeStruct((b, n, k), jnp.int32),
    )(score)


def _mrconv_body(score_ref, yt_ref, xft_ref, o_ref, *, k, dil):
    score = score_ref[0]
    n, ny = score.shape
    lane = lax.broadcasted_iota(jnp.int32, (n, ny), 1).astype(jnp.float32)
    neg_inf = jnp.float32(-jnp.inf)
    yt = yt_ref[0]
    xmax = None
    for m in range(k * dil):
        cmax = jnp.max(score, axis=1, keepdims=True)
        hit = score == cmax
        idx = jnp.min(jnp.where(hit, lane, jnp.float32(ny)), axis=1,
                      keepdims=True)
        sel = lane == idx
        if m % dil == 0:
            onehot = sel.astype(jnp.float32)
            xj = jnp.dot(onehot, yt, precision=lax.Precision.HIGHEST,
                         preferred_element_type=jnp.float32)
            xmax = xj if xmax is None else jnp.maximum(xmax, xj)
        if m < k * dil - 1:
            score = jnp.where(sel, neg_inf, score)
    o_ref[0] = xmax - xft_ref[0]


def _knn_mrconv(score, y_t, xf_t, k, dil):
    b, n, ny = score.shape
    c = y_t.shape[2]
    return pl.pallas_call(
        functools.partial(_mrconv_body, k=k, dil=dil),
        grid=(b,),
        in_specs=[pl.BlockSpec((1, n, ny), lambda i: (i, 0, 0)),
                  pl.BlockSpec((1, ny, c), lambda i: (i, 0, 0)),
                  pl.BlockSpec((1, n, c), lambda i: (i, 0, 0))],
        out_specs=pl.BlockSpec((1, n, c), lambda i: (i, 0, 0)),
        out_shape=jax.ShapeDtypeStruct((b, n, c), jnp.float32),
    )(score, y_t, xf_t)


def _bn(x, g, b, eps=_EPS):
    mean = x.mean(axis=(0, 2, 3), keepdims=True)
    var = x.var(axis=(0, 2, 3), keepdims=True)
    xn = (x - mean) / jnp.sqrt(var + eps)
    return xn * g[None, :, None, None] + b[None, :, None, None]


def _conv1x1(x, w, b):
    return jnp.einsum("bihw,oi->bohw", x, w) + b[None, :, None, None]


def _down_blk(x, p):
    out = lax.conv_general_dilated(x, p["w"], (2, 2), ((1, 1), (1, 1)),
                                   dimension_numbers=("NCHW", "OIHW", "NCHW"))
    out = out + p["b"][None, :, None, None]
    return _bn(out, p["g"], p["bt"])


def _ffn_blk(x, p):
    s = x
    x = _bn(_conv1x1(x, p["fc1_w"], p["fc1_b"]), p["fc1_g"], p["fc1_bt"])
    x = jax.nn.relu(x)
    x = _bn(_conv1x1(x, p["fc2_w"], p["fc2_b"]), p["fc2_g"], p["fc2_bt"])
    return x + s


def _grapher_blk(x, p, k, dil, r, use_pallas):
    b, c, h, w = x.shape
    shortcut = x
    x = _bn(_conv1x1(x, p["fc1_w"], p["fc1_b"]), p["fc1_g"], p["fc1_bt"])
    n = h * w
    xf = x.reshape(b, c, n)
    if r > 1:
        y = x.reshape(b, c, h // r, r, w // r, r).mean(axis=(3, 5)).reshape(b, c, -1)
    else:
        y = xf
    xn = xf / (jnp.linalg.norm(xf, axis=1, keepdims=True) + 1e-12)
    yn = y / (jnp.linalg.norm(y, axis=1, keepdims=True) + 1e-12)
    dist = (jnp.sum(xn * xn, axis=1)[:, :, None]
            - 2.0 * jnp.einsum("bcn,bcm->bnm", xn, yn)
            + jnp.sum(yn * yn, axis=1)[:, None, :])
    score = -(dist + p["relpos"][None, :, :])
    if use_pallas:
        m_t = _knn_mrconv(score, y.transpose(0, 2, 1),
                          xf.transpose(0, 2, 1), k, dil)
        m = m_t.transpose(0, 2, 1)[:, :, :, None]
        xi = xf[:, :, :, None]
    else:
        _, nn_idx = jax.lax.top_k(score, k * dil)
        nn_idx = nn_idx[:, :, ::dil]
        idx = jnp.broadcast_to(nn_idx.reshape(b, 1, n * k), (b, c, n * k))
        xj = jnp.take_along_axis(y, idx, axis=2).reshape(b, c, n, k)
        xi = xf[:, :, :, None]
        m = jnp.max(xj - xi, axis=3, keepdims=True)
    z = jnp.concatenate([xi[:, :, None, :, :], m[:, :, None, :, :]],
                        axis=2).reshape(b, 2 * c, n, 1)
    z = _conv1x1(z, p["nn_w"], p["nn_b"])
    z = _bn(z, p["nn_g"], p["nn_bt"])
    z = jax.nn.relu(z)
    z = z.reshape(b, 2 * c, h, w)
    z = _bn(_conv1x1(z, p["fc2_w"], p["fc2_b"]), p["fc2_g"], p["fc2_bt"])
    return z + shortcut


def kernel(x, h0, params):
    del h0
    specs = _layer_specs()
    n_b = _BLOCKS
    out_at = [2 * n_b[0] - 1,
              2 * (n_b[0] + n_b[1]),
              2 * (n_b[0] + n_b[1] + n_b[2]) + 1,
              2 * (n_b[0] + n_b[1] + n_b[2] + n_b[3]) + 2]
    outs = []
    gi = 0
    for i, (spec, p) in enumerate(zip(specs, params["layers"])):
        if spec[0] == "down":
            x = _down_blk(x, p)
        elif spec[0] == "grapher":
            x = _grapher_blk(x, p, spec[1], spec[2], spec[3], gi in (0, 1))
            gi += 1
        else:
            x = _ffn_blk(x, p)
        if i in out_at:
            outs.append(x)
    return tuple(outs)
